# baseline (speedup 1.0000x reference)
import functools

import jax
import jax.numpy as jnp
from jax import lax
from jax.experimental import pallas as pl
from jax.experimental.pallas import tpu as pltpu
from jax.experimental.pallas import tpu_sc as plsc

NC = 2
NS = 16
L = 16
NW = NC * NS

MAXQ = 31
LUT_SIZE = MAXQ + 1


def _build_sc_call(N, K, R, unroll):
    rows_per_w = N // NW
    nblk = rows_per_w // R
    mesh = plsc.VectorSubcoreMesh(
        core_axis_name="c", subcore_axis_name="s", num_cores=NC,
        num_subcores=NS)

    @functools.partial(
        pl.kernel,
        out_type=jax.ShapeDtypeStruct((N, K), jnp.float32),
        mesh=mesh,
        compiler_params=pltpu.CompilerParams(needs_layout_passes=False),
        scratch_types=dict(
            lut_v=pltpu.VMEM((LUT_SIZE,), jnp.float32),
            sc_v=pltpu.VMEM((rows_per_w,), jnp.float32),
            rs_v=pltpu.VMEM((rows_per_w,), jnp.float32),
            zr_v=pltpu.VMEM((rows_per_w,), jnp.float32),
            dql=[pltpu.VMEM((LUT_SIZE,), jnp.float32) for _ in range(2)],
            inb=[pltpu.VMEM((R, K), jnp.float32) for _ in range(2)],
            outb=[pltpu.VMEM((R, K), jnp.float32) for _ in range(2)],
            insem=[pltpu.SemaphoreType.DMA for _ in range(2)],
            outsem=[pltpu.SemaphoreType.DMA for _ in range(2)],
        ),
    )
    def sc_quant(x_hbm, scale_hbm, rscale_hbm, zero_hbm, lut_hbm, out_hbm,
                 *, lut_v, sc_v, rs_v, zr_v, dql, inb, outb, insem, outsem):
        wid = lax.axis_index("s") * NC + lax.axis_index("c")
        base_row = wid * rows_per_w

        pltpu.sync_copy(scale_hbm.at[pl.ds(base_row, rows_per_w)], sc_v)
        pltpu.sync_copy(rscale_hbm.at[pl.ds(base_row, rows_per_w)], rs_v)
        pltpu.sync_copy(zero_hbm.at[pl.ds(base_row, rows_per_w)], zr_v)
        pltpu.sync_copy(lut_hbm, lut_v)

        def bcast(ref, i):
            idx = jnp.full((L,), i, dtype=jnp.int32)
            return plsc.load_gather(ref, [idx])

        lut_lo = lut_v[pl.ds(0, L)]
        lut_hi = lut_v[pl.ds(L, L)]

        def in_start(blk, s):
            pltpu.async_copy(
                x_hbm.at[pl.ds(base_row + blk * R, R)], inb[s], insem[s])

        def in_wait(s):
            pltpu.make_async_copy(
                x_hbm.at[pl.ds(0, R)], inb[s], insem[s]).wait()

        def out_start(blk, s):
            pltpu.async_copy(
                outb[s], out_hbm.at[pl.ds(base_row + blk * R, R)],
                outsem[s])

        def out_wait(s):
            pltpu.make_async_copy(
                outb[s], out_hbm.at[pl.ds(0, R)], outsem[s]).wait()

        in_start(0, 0)
        in_start(1, 1)

        @pl.loop(0, nblk, step=2)
        def _pair(bp):
            for s in range(2):
                blk = bp + s
                in_wait(s)

                @pl.when(bp >= 2)
                def _():
                    out_wait(s)

                for r in range(R):
                    row_local = blk * R + r
                    sv = bcast(sc_v, row_local)
                    zv = bcast(zr_v, row_local)
                    rsv = bcast(rs_v, row_local)
                    cv = 0.5 - zv * rsv
                    dq_ref = dql[r % 2]
                    dq_ref[pl.ds(0, L)] = lut_lo * sv + zv
                    dq_ref[pl.ds(L, L)] = lut_hi * sv + zv

                    @plsc.parallel_loop(0, K // L, 1, unroll=unroll)
                    def _chunk(c):
                        xv = inb[s][r, pl.ds(c * L, L)]
                        t = xv * rsv + cv
                        t = jnp.minimum(jnp.maximum(t, 0.5), MAXQ + 0.5)
                        qi = t.astype(jnp.int32)
                        outb[s][r, pl.ds(c * L, L)] = (
                            plsc.load_gather(dq_ref, [qi]))

                out_start(blk, s)

                @pl.when(blk + 2 < nblk)
                def _():
                    in_start(blk + 2, s)

        out_wait(0)
        out_wait(1)

    return sc_quant


@functools.lru_cache(maxsize=None)
def _get_call(N, K):
    return jax.jit(_build_sc_call(N, K, R=4, unroll=8))


def kernel(x, scale, zero, choice_bits):
    scale = scale.astype(jnp.float32)
    rscale = 1.0 / scale
    cb = choice_bits.astype(jnp.float32)
    grid = jnp.arange(LUT_SIZE, dtype=jnp.float32)
    lut = jnp.broadcast_to(cb[0], grid.shape)
    best = jnp.abs(grid - cb[0])
    for j in range(1, cb.shape[0]):
        dj = jnp.abs(grid - cb[j])
        lut = jnp.where(dj < best, cb[j], lut)
        best = jnp.minimum(dj, best)
    call = _get_call(x.shape[0], x.shape[1])
    return call(x.astype(jnp.float32), scale, rscale,
                zero.astype(jnp.float32), lut)


# speedup vs baseline: 1.0161x; 1.0161x over previous
import functools

import jax
import jax.numpy as jnp
from jax import lax
from jax.experimental import pallas as pl
from jax.experimental.pallas import tpu as pltpu
from jax.experimental.pallas import tpu_sc as plsc

NC = 2
NS = 16
L = 16
NW = NC * NS

MAXQ = 31
LUT_SIZE = MAXQ + 1
MAGIC = 2.0 ** 23


def _build_sc_call(N, K, R, unroll):
    rows_per_w = N // NW
    nblk = rows_per_w // R
    mesh = plsc.VectorSubcoreMesh(
        core_axis_name="c", subcore_axis_name="s", num_cores=NC,
        num_subcores=NS)

    @functools.partial(
        pl.kernel,
        out_type=jax.ShapeDtypeStruct((N, K), jnp.float32),
        mesh=mesh,
        compiler_params=pltpu.CompilerParams(needs_layout_passes=False),
        scratch_types=dict(
            rs_v=pltpu.VMEM((rows_per_w,), jnp.float32),
            rc_v=pltpu.VMEM((rows_per_w,), jnp.float32),
            dq_v=pltpu.VMEM((rows_per_w, LUT_SIZE), jnp.float32),
            inb=[pltpu.VMEM((R, K), jnp.float32) for _ in range(2)],
            outb=[pltpu.VMEM((R, K), jnp.float32) for _ in range(2)],
            insem=[pltpu.SemaphoreType.DMA for _ in range(2)],
            outsem=[pltpu.SemaphoreType.DMA for _ in range(2)],
        ),
    )
    def sc_quant(x_hbm, rscale_hbm, rowc_hbm, dqlut_hbm, out_hbm,
                 *, rs_v, rc_v, dq_v, inb, outb, insem, outsem):
        wid = lax.axis_index("s") * NC + lax.axis_index("c")
        base_row = wid * rows_per_w

        pltpu.sync_copy(rscale_hbm.at[pl.ds(base_row, rows_per_w)], rs_v)
        pltpu.sync_copy(rowc_hbm.at[pl.ds(base_row, rows_per_w)], rc_v)
        pltpu.sync_copy(dqlut_hbm.at[pl.ds(base_row, rows_per_w)], dq_v)

        def bcast(ref, i):
            idx = jnp.full((L,), i, dtype=jnp.int32)
            return plsc.load_gather(ref, [idx])

        def in_start(blk, s):
            pltpu.async_copy(
                x_hbm.at[pl.ds(base_row + blk * R, R)], inb[s], insem[s])

        def in_wait(s):
            pltpu.make_async_copy(
                x_hbm.at[pl.ds(0, R)], inb[s], insem[s]).wait()

        def out_start(blk, s):
            pltpu.async_copy(
                outb[s], out_hbm.at[pl.ds(base_row + blk * R, R)],
                outsem[s])

        def out_wait(s):
            pltpu.make_async_copy(
                outb[s], out_hbm.at[pl.ds(0, R)], outsem[s]).wait()

        in_start(0, 0)
        in_start(1, 1)

        @pl.loop(0, nblk, step=2)
        def _pair(bp):
            for s in range(2):
                blk = bp + s
                in_wait(s)

                @pl.when(bp >= 2)
                def _():
                    out_wait(s)

                for r in range(R):
                    row_local = blk * R + r
                    rsv = bcast(rs_v, row_local)
                    cv = bcast(rc_v, row_local)
                    dq_row = dq_v.at[row_local]

                    @plsc.parallel_loop(0, K // L, 1, unroll=unroll)
                    def _chunk(c):
                        xv = inb[s][r, pl.ds(c * L, L)]
                        t = xv * rsv + cv
                        t = jnp.minimum(jnp.maximum(t, 0.0), float(MAXQ))
                        y = lax.bitcast_convert_type(t + MAGIC, jnp.int32)
                        qi = jnp.bitwise_and(y, MAXQ)
                        outb[s][r, pl.ds(c * L, L)] = (
                            plsc.load_gather(dq_row, [qi]))

                out_start(blk, s)

                @pl.when(blk + 2 < nblk)
                def _():
                    in_start(blk + 2, s)

        out_wait(0)
        out_wait(1)

    return sc_quant


@functools.lru_cache(maxsize=None)
def _get_call(N, K):
    return jax.jit(_build_sc_call(N, K, R=4, unroll=8))


def kernel(x, scale, zero, choice_bits):
    scale = scale.astype(jnp.float32)
    zero = zero.astype(jnp.float32)
    rscale = 1.0 / scale
    rowc = -zero * rscale
    cb = choice_bits.astype(jnp.float32)
    grid = jnp.arange(LUT_SIZE, dtype=jnp.float32)
    lut = jnp.broadcast_to(cb[0], grid.shape)
    best = jnp.abs(grid - cb[0])
    for j in range(1, cb.shape[0]):
        dj = jnp.abs(grid - cb[j])
        lut = jnp.where(dj < best, cb[j], lut)
        best = jnp.minimum(dj, best)
    dqlut = scale[:, None] * lut[None, :] + zero[:, None]
    call = _get_call(x.shape[0], x.shape[1])
    return call(x.astype(jnp.float32), rscale, rowc, dqlut)
